# baseline (speedup 1.0000x reference)
import functools

import jax
import jax.numpy as jnp
from jax import lax
from jax.experimental import pallas as pl
from jax.experimental.pallas import tpu as pltpu
from jax.experimental.pallas import tpu_sc as plsc

B = 64
V = 1_000_000
LANES = 128
CHUNK = 16384
NB = (V + CHUNK - 1) // CHUNK
K = CHUNK // LANES

SC_L = 16


def _partials_body(gt_ref, xtok_ref, x_ref, out_ref, m_acc, s_acc):
    j = pl.program_id(0)
    nb = pl.num_programs(0)

    @pl.when(j == 0)
    def _init():
        m_acc[...] = jnp.full((B, LANES), -jnp.inf, jnp.float32)
        s_acc[...] = jnp.zeros((B, LANES), jnp.float32)

    inv = 1.0 / gt_ref[0, 0]

    def update(xb):
        bm = jnp.max(xb, axis=1)
        m_old = m_acc[...]
        m_new = jnp.maximum(m_old, bm)
        e = jnp.exp((xb - m_new[:, None, :]) * inv)
        bs = jnp.sum(e, axis=1)
        s_acc[...] = s_acc[...] * jnp.exp((m_old - m_new) * inv) + bs
        m_acc[...] = m_new

    @pl.when(j < nb - 1)
    def _fast():
        update(x_ref[...].reshape(B, K, LANES))

    @pl.when(j == nb - 1)
    def _tail():
        xb = x_ref[...].reshape(B, K, LANES)
        col = (
            lax.broadcasted_iota(jnp.int32, (B, K, LANES), 1) * LANES
            + lax.broadcasted_iota(jnp.int32, (B, K, LANES), 2)
            + j * CHUNK
        )
        update(jnp.where(col < V, xb, -jnp.inf))
        mf = jnp.max(m_acc[...], axis=1, keepdims=True)
        sf = jnp.sum(
            s_acc[...] * jnp.exp((m_acc[...] - mf) * inv), axis=1, keepdims=True
        )
        res = jnp.exp((xtok_ref[...] - mf) * inv) / sf
        out_ref[...] = jnp.broadcast_to(res, (B, LANES))


def _softmax_gathered(x, gt, xtok):
    return pl.pallas_call(
        _partials_body,
        grid=(NB,),
        in_specs=[
            pl.BlockSpec(memory_space=pltpu.SMEM),
            pl.BlockSpec((B, 1), lambda j: (0, 0)),
            pl.BlockSpec((B, CHUNK), lambda j: (0, j)),
        ],
        out_specs=pl.BlockSpec((B, LANES), lambda j: (0, 0)),
        out_shape=jax.ShapeDtypeStruct((B, LANES), jnp.float32),
        scratch_shapes=[
            pltpu.VMEM((B, LANES), jnp.float32),
            pltpu.VMEM((B, LANES), jnp.float32),
        ],
        compiler_params=pltpu.CompilerParams(
            dimension_semantics=("arbitrary",),
        ),
    )(gt, xtok, x)


def _make_sc_gather():
    mesh = plsc.VectorSubcoreMesh(core_axis_name="c", subcore_axis_name="s")

    @functools.partial(
        pl.kernel,
        mesh=mesh,
        out_type=jax.ShapeDtypeStruct((B,), jnp.float32),
        scratch_types=[
            pltpu.VMEM((B,), jnp.int32),
            pltpu.VMEM((B,), jnp.int32),
            pltpu.VMEM((B,), jnp.float32),
            pltpu.SemaphoreType.DMA,
        ],
    )
    def sc_gather(xf_hbm, tok_hbm, out_hbm, tok_v, idx_v, val_v, sem):
        wid = lax.axis_index("s") * 2 + lax.axis_index("c")

        @pl.when(wid == 0)
        def _():
            pltpu.sync_copy(tok_hbm, tok_v)
            iota = lax.iota(jnp.int32, SC_L)
            for g in range(B // SC_L):
                tok = tok_v[pl.ds(g * SC_L, SC_L)]
                idx_v[pl.ds(g * SC_L, SC_L)] = (iota + g * SC_L) * V + tok
            pltpu.async_copy(xf_hbm.at[idx_v], val_v, sem).wait()
            pltpu.sync_copy(val_v, out_hbm)

    return sc_gather


_SC_CACHE = []


def _sc_gather_call(*args):
    if not _SC_CACHE:
        _SC_CACHE.append(_make_sc_gather())
    return _SC_CACHE[0](*args)


def kernel(x, tokens, general_temp, top_temp):
    del top_temp
    gt = jnp.reshape(general_temp, (1, 1)).astype(jnp.float32)
    xtok = _sc_gather_call(x.reshape(B * V), tokens.astype(jnp.int32))
    out2d = _softmax_gathered(x, gt, xtok.reshape(B, 1))
    return out2d[:, 0]
